# baseline (speedup 1.0000x reference)
import functools

import jax
import jax.numpy as jnp
from jax import lax
from jax.experimental import pallas as pl
from jax.experimental.pallas import tpu as pltpu
from jax.experimental.pallas import tpu_sc as plsc

_C = 26
_V = 100000
_D = 32
_B = 16384
_NW = 32
_L = 16

_BLK = 512
_NB = _B // _BLK
_G_TASKS = _C * _NB
_G_ITERS = _G_TASKS // _NW

_mesh = plsc.VectorSubcoreMesh(core_axis_name="c", subcore_axis_name="s")


@functools.partial(
    pl.kernel,
    mesh=_mesh,
    out_type=jax.ShapeDtypeStruct((_C, _D, _B), jnp.float32),
    scratch_types=[
        pltpu.VMEM((_BLK,), jnp.int32),
        pltpu.VMEM((_BLK,), jnp.int32),
        pltpu.VMEM((_BLK, 128), jnp.float32),
        pltpu.VMEM((_D, _BLK), jnp.float32),
        pltpu.SemaphoreType.DMA,
    ],
    compiler_params=pltpu.CompilerParams(
        use_tc_tiling_on_sc=False, needs_layout_passes=False),
)
def _gather_all(idx_hbm, wide_hbm, out_hbm, idx_v, row_v, rows_v, out_vT, sem):
    wid = lax.axis_index("s") * 2 + lax.axis_index("c")

    def task(t, carry):
        tid = wid + t * _NW
        cat = tid // _NB
        b0 = (tid % _NB) * _BLK
        pltpu.sync_copy(idx_hbm.at[cat, pl.ds(b0, _BLK)], idx_v)
        base = cat * (_V // 4)

        def prep(j, c2):
            v = idx_v[pl.ds(j * _L, _L)]
            row_v[pl.ds(j * _L, _L)] = base + lax.shift_right_logical(v, 2)
            return c2

        lax.fori_loop(0, _BLK // _L, prep, 0)
        pltpu.async_copy(wide_hbm.at[row_v], rows_v, sem).wait()

        def grp(j, c2):
            sub = idx_v[pl.ds(j * _L, _L)]
            rvec = lax.iota(jnp.int32, _L) + j * _L
            cvec = lax.shift_left(jnp.bitwise_and(sub, 3), 5)

            def dloop(d, c3):
                out_vT[d, pl.ds(j * _L, _L)] = plsc.load_gather(
                    rows_v, [rvec, cvec + d])
                return c3

            lax.fori_loop(0, _D, dloop, 0)
            return c2

        lax.fori_loop(0, _BLK // _L, grp, 0)
        pltpu.sync_copy(out_vT, out_hbm.at[cat, :, pl.ds(b0, _BLK)])
        return carry

    lax.fori_loop(0, _G_ITERS, task, 0)


def kernel(input, tables):
    idx_t = input.astype(jnp.int32).T
    wide = jnp.reshape(tables, (_C * _V // 4, 128))
    out = _gather_all(idx_t, wide)
    return jnp.transpose(out, (2, 0, 1))


# speedup vs baseline: 1.5235x; 1.5235x over previous
import functools

import jax
import jax.numpy as jnp
from jax import lax
from jax.experimental import pallas as pl
from jax.experimental.pallas import tpu as pltpu
from jax.experimental.pallas import tpu_sc as plsc

_C = 26
_V = 100000
_D = 32
_B = 16384
_NW = 32

_mesh = plsc.VectorSubcoreMesh(core_axis_name="c", subcore_axis_name="s")


@functools.partial(
    pl.kernel,
    mesh=_mesh,
    out_type=jax.ShapeDtypeStruct((_C, _D, _B), jnp.float32),
    scratch_types=[
        pltpu.VMEM((_B,), jnp.int32),
        pltpu.VMEM((_B,), jnp.float32),
        pltpu.SemaphoreType.DMA,
    ],
    compiler_params=pltpu.CompilerParams(
        use_tc_tiling_on_sc=False, needs_layout_passes=False),
)
def _gather_all(idx_hbm, tab_hbm, out_hbm, idx_v, col_v, sem):
    wid = lax.axis_index("s") * 2 + lax.axis_index("c")

    def task(c, carry):
        pltpu.sync_copy(idx_hbm.at[c], idx_v)
        pltpu.async_copy(tab_hbm.at[c, wid].at[idx_v], col_v, sem).wait()
        pltpu.sync_copy(col_v, out_hbm.at[c, wid])
        return carry

    lax.fori_loop(0, _C, task, 0)


def kernel(input, tables):
    idx_t = input.astype(jnp.int32).T
    tt = jnp.transpose(tables, (0, 2, 1))
    out = _gather_all(idx_t, tt)
    return jnp.transpose(out, (2, 0, 1))


# speedup vs baseline: 1.6117x; 1.0579x over previous
import functools

import jax
import jax.numpy as jnp
from jax import lax
from jax.experimental import pallas as pl
from jax.experimental.pallas import tpu as pltpu
from jax.experimental.pallas import tpu_sc as plsc

_C = 26
_V = 100000
_D = 32
_B = 16384
_NW = 32

_mesh = plsc.VectorSubcoreMesh(core_axis_name="c", subcore_axis_name="s")


@functools.partial(
    pl.kernel,
    mesh=_mesh,
    out_type=jax.ShapeDtypeStruct((_C, _D, _B), jnp.float32),
    scratch_types=[
        pltpu.VMEM((_B,), jnp.int32),
        pltpu.VMEM((_B,), jnp.int32),
        pltpu.VMEM((_B,), jnp.float32),
        pltpu.VMEM((_B,), jnp.float32),
        pltpu.SemaphoreType.DMA,
        pltpu.SemaphoreType.DMA,
    ],
    compiler_params=pltpu.CompilerParams(
        use_tc_tiling_on_sc=False, needs_layout_passes=False),
)
def _gather_all(idx_hbm, tab_hbm, out_hbm, idx_v0, idx_v1, col_v0, col_v1,
                sem0, sem1):
    wid = lax.axis_index("s") * 2 + lax.axis_index("c")
    idx_v = (idx_v0, idx_v1)
    col_v = (col_v0, col_v1)
    sem = (sem0, sem1)

    def pair(t, carry):
        for par in (0, 1):
            c = t * 2 + par

            @pl.when(c >= 2)
            def _():
                pltpu.make_async_copy(
                    out_hbm.at[0, 0], col_v[par], sem[par]).wait()
                pltpu.sync_copy(col_v[par], out_hbm.at[c - 2, wid])

            pltpu.sync_copy(idx_hbm.at[c], idx_v[par])
            pltpu.async_copy(
                tab_hbm.at[c, wid].at[idx_v[par]], col_v[par], sem[par])
        return carry

    lax.fori_loop(0, _C // 2, pair, 0)
    for par in (0, 1):
        pltpu.make_async_copy(out_hbm.at[0, 0], col_v[par], sem[par]).wait()
        pltpu.sync_copy(col_v[par], out_hbm.at[_C - 2 + par, wid])


def kernel(input, tables):
    idx_t = input.astype(jnp.int32).T
    tt = jnp.transpose(tables, (0, 2, 1))
    out = _gather_all(idx_t, tt)
    return jnp.transpose(out, (2, 0, 1))
